# baseline (speedup 1.0000x reference)
import functools

import jax
import jax.numpy as jnp
from jax import lax
from jax.experimental import pallas as pl
from jax.experimental.pallas import tpu as pltpu, tpu_sc as plsc

USER_NUM = 1000000
ITEM_NUM = 1000000
FACTOR = 64
BATCH = 16384

_NC = 2
_NS = 16
_NW = _NC * _NS
_BPW = BATCH // _NW
_CHUNK = 128
_NCHUNK = _BPW // _CHUNK


def _pairmf_body(user_hbm, item_i_hbm, item_j_hbm, eu_hbm, ei_hbm,
                 pred_i_hbm, pred_j_hbm,
                 idx_u, idx_i, idx_j, rows_u, rows_i, rows_j,
                 out_i, out_j, sem_u, sem_i, sem_j):
    wid = lax.axis_index("s") * _NC + lax.axis_index("c")
    base = wid * _BPW

    for c in range(_NCHUNK):
        pltpu.sync_copy(user_hbm.at[pl.ds(base + c * _CHUNK, _CHUNK)], idx_u.at[c])
        pltpu.sync_copy(item_i_hbm.at[pl.ds(base + c * _CHUNK, _CHUNK)], idx_i.at[c])
        pltpu.sync_copy(item_j_hbm.at[pl.ds(base + c * _CHUNK, _CHUNK)], idx_j.at[c])

    for c in range(_NCHUNK):
        cp_u = pltpu.async_copy(eu_hbm.at[idx_u.at[c]], rows_u, sem_u)
        cp_i = pltpu.async_copy(ei_hbm.at[idx_i.at[c]], rows_i, sem_i)
        cp_j = pltpu.async_copy(ei_hbm.at[idx_j.at[c]], rows_j, sem_j)
        cp_u.wait()
        cp_i.wait()
        cp_j.wait()

        lane = lax.iota(jnp.int32, 16)

        def group(g, _):
            rvec = g * 16 + lane
            acc_i = [jnp.zeros((16,), jnp.float32) for _ in range(4)]
            acc_j = [jnp.zeros((16,), jnp.float32) for _ in range(4)]
            for d in range(FACTOR):
                dcol = jnp.full((16,), d, jnp.int32)
                u = plsc.load_gather(rows_u, [rvec, dcol])
                vi = plsc.load_gather(rows_i, [rvec, dcol])
                vj = plsc.load_gather(rows_j, [rvec, dcol])
                k = d % 4
                acc_i[k] = acc_i[k] + u * vi
                acc_j[k] = acc_j[k] + u * vj
            out_i[pl.ds(c * _CHUNK + g * 16, 16)] = (
                (acc_i[0] + acc_i[1]) + (acc_i[2] + acc_i[3]))
            out_j[pl.ds(c * _CHUNK + g * 16, 16)] = (
                (acc_j[0] + acc_j[1]) + (acc_j[2] + acc_j[3]))
            return 0

        lax.fori_loop(0, _CHUNK // 16, group, 0)

    pltpu.sync_copy(out_i, pred_i_hbm.at[pl.ds(base, _BPW)])
    pltpu.sync_copy(out_j, pred_j_hbm.at[pl.ds(base, _BPW)])


@jax.jit
def _pairmf(user, item_i, item_j, embed_user, embed_item):
    mesh = plsc.VectorSubcoreMesh(core_axis_name="c", subcore_axis_name="s")
    f = functools.partial(
        pl.kernel, mesh=mesh,
        compiler_params=pltpu.CompilerParams(
            needs_layout_passes=False, use_tc_tiling_on_sc=False),
        out_type=(jax.ShapeDtypeStruct((BATCH,), jnp.float32),
                  jax.ShapeDtypeStruct((BATCH,), jnp.float32)),
        scratch_types=[
            pltpu.VMEM((_NCHUNK, _CHUNK), jnp.int32),
            pltpu.VMEM((_NCHUNK, _CHUNK), jnp.int32),
            pltpu.VMEM((_NCHUNK, _CHUNK), jnp.int32),
            pltpu.VMEM((_CHUNK, FACTOR), jnp.float32),
            pltpu.VMEM((_CHUNK, FACTOR), jnp.float32),
            pltpu.VMEM((_CHUNK, FACTOR), jnp.float32),
            pltpu.VMEM((_BPW,), jnp.float32),
            pltpu.VMEM((_BPW,), jnp.float32),
            pltpu.SemaphoreType.DMA,
            pltpu.SemaphoreType.DMA,
            pltpu.SemaphoreType.DMA,
        ],
    )(_pairmf_body)
    return f(user, item_i, item_j, embed_user, embed_item)


def kernel(user, item_i, item_j, embed_user, embed_item):
    user = user.astype(jnp.int32)
    item_i = item_i.astype(jnp.int32)
    item_j = item_j.astype(jnp.int32)
    return _pairmf(user, item_i, item_j, embed_user, embed_item)


# speedup vs baseline: 1.0315x; 1.0315x over previous
import functools

import jax
import jax.numpy as jnp
from jax import lax
from jax.experimental import pallas as pl
from jax.experimental.pallas import tpu as pltpu, tpu_sc as plsc

USER_NUM = 1000000
ITEM_NUM = 1000000
FACTOR = 64
BATCH = 16384

_NC = 2
_NS = 16
_NW = _NC * _NS
_BPW = BATCH // _NW
_CHUNK = 128
_NCHUNK = _BPW // _CHUNK


def _pairmf_body(user_hbm, item_i_hbm, item_j_hbm, eu_hbm, ei_hbm,
                 pred_i_hbm, pred_j_hbm,
                 idx_u, idx_i, idx_j, g_u, g_i, g_j,
                 rows_u, rows_i, rows_j,
                 out_i, out_j, sem_u, sem_i, sem_j):
    wid = lax.axis_index("s") * _NC + lax.axis_index("c")
    base = wid * _BPW
    lane = lax.iota(jnp.int32, 16)

    for c in range(_NCHUNK):
        pltpu.sync_copy(user_hbm.at[pl.ds(base + c * _CHUNK, _CHUNK)], idx_u.at[c])
        pltpu.sync_copy(item_i_hbm.at[pl.ds(base + c * _CHUNK, _CHUNK)], idx_i.at[c])
        pltpu.sync_copy(item_j_hbm.at[pl.ds(base + c * _CHUNK, _CHUNK)], idx_j.at[c])
    for c in range(_NCHUNK):
        for s in range(_CHUNK // 16):
            sl = pl.ds(s * 16, 16)
            g_u[c, sl] = lax.shift_right_logical(idx_u[c, sl], 1)
            g_i[c, sl] = lax.shift_right_logical(idx_i[c, sl], 1)
            g_j[c, sl] = lax.shift_right_logical(idx_j[c, sl], 1)

    for c in range(_NCHUNK):
        cp_u = pltpu.async_copy(eu_hbm.at[g_u.at[c]], rows_u, sem_u)
        cp_i = pltpu.async_copy(ei_hbm.at[g_i.at[c]], rows_i, sem_i)
        cp_j = pltpu.async_copy(ei_hbm.at[g_j.at[c]], rows_j, sem_j)
        cp_u.wait()
        cp_i.wait()
        cp_j.wait()

        def group(g, _):
            rvec = g * 16 + lane
            sl = pl.ds(g * 16, 16)
            off_u = lax.shift_left(idx_u[c, sl] & 1, 6)
            off_i = lax.shift_left(idx_i[c, sl] & 1, 6)
            off_j = lax.shift_left(idx_j[c, sl] & 1, 6)
            acc_i = [jnp.zeros((16,), jnp.float32) for _ in range(4)]
            acc_j = [jnp.zeros((16,), jnp.float32) for _ in range(4)]
            for d in range(FACTOR):
                e = (lane + d) & 63
                u = plsc.load_gather(rows_u, [rvec, off_u + e])
                vi = plsc.load_gather(rows_i, [rvec, off_i + e])
                vj = plsc.load_gather(rows_j, [rvec, off_j + e])
                k = d % 4
                acc_i[k] = acc_i[k] + u * vi
                acc_j[k] = acc_j[k] + u * vj
            out_i[pl.ds(c * _CHUNK + g * 16, 16)] = (
                (acc_i[0] + acc_i[1]) + (acc_i[2] + acc_i[3]))
            out_j[pl.ds(c * _CHUNK + g * 16, 16)] = (
                (acc_j[0] + acc_j[1]) + (acc_j[2] + acc_j[3]))
            return 0

        lax.fori_loop(0, _CHUNK // 16, group, 0)

    pltpu.sync_copy(out_i, pred_i_hbm.at[pl.ds(base, _BPW)])
    pltpu.sync_copy(out_j, pred_j_hbm.at[pl.ds(base, _BPW)])


@jax.jit
def _pairmf(user, item_i, item_j, embed_user, embed_item):
    mesh = plsc.VectorSubcoreMesh(core_axis_name="c", subcore_axis_name="s")
    f = functools.partial(
        pl.kernel, mesh=mesh,
        compiler_params=pltpu.CompilerParams(
            needs_layout_passes=False, use_tc_tiling_on_sc=True),
        out_type=(jax.ShapeDtypeStruct((BATCH,), jnp.float32),
                  jax.ShapeDtypeStruct((BATCH,), jnp.float32)),
        scratch_types=[
            pltpu.VMEM((_NCHUNK, _CHUNK), jnp.int32),
            pltpu.VMEM((_NCHUNK, _CHUNK), jnp.int32),
            pltpu.VMEM((_NCHUNK, _CHUNK), jnp.int32),
            pltpu.VMEM((_NCHUNK, _CHUNK), jnp.int32),
            pltpu.VMEM((_NCHUNK, _CHUNK), jnp.int32),
            pltpu.VMEM((_NCHUNK, _CHUNK), jnp.int32),
            pltpu.VMEM((_CHUNK, 2 * FACTOR), jnp.float32),
            pltpu.VMEM((_CHUNK, 2 * FACTOR), jnp.float32),
            pltpu.VMEM((_CHUNK, 2 * FACTOR), jnp.float32),
            pltpu.VMEM((_BPW,), jnp.float32),
            pltpu.VMEM((_BPW,), jnp.float32),
            pltpu.SemaphoreType.DMA,
            pltpu.SemaphoreType.DMA,
            pltpu.SemaphoreType.DMA,
        ],
    )(_pairmf_body)
    eu2 = embed_user.reshape(USER_NUM // 2, 2 * FACTOR)
    ei2 = embed_item.reshape(ITEM_NUM // 2, 2 * FACTOR)
    return f(user, item_i, item_j, eu2, ei2)


def kernel(user, item_i, item_j, embed_user, embed_item):
    user = user.astype(jnp.int32)
    item_i = item_i.astype(jnp.int32)
    item_j = item_j.astype(jnp.int32)
    return _pairmf(user, item_i, item_j, embed_user, embed_item)


# speedup vs baseline: 1.0883x; 1.0551x over previous
import functools

import jax
import jax.numpy as jnp
from jax import lax
from jax.experimental import pallas as pl
from jax.experimental.pallas import tpu as pltpu, tpu_sc as plsc

USER_NUM = 1000000
ITEM_NUM = 1000000
FACTOR = 64
BATCH = 16384

_NC = 2
_NS = 16
_NW = _NC * _NS
_BPW = BATCH // _NW
_CHUNK = 128
_NCHUNK = _BPW // _CHUNK


def _pairmf_body(user_hbm, item_i_hbm, item_j_hbm, eu_hbm, ei_hbm,
                 pred_i_hbm, pred_j_hbm,
                 idx_u, idx_i, idx_j,
                 rows_u, rows_i, rows_j,
                 out_i, out_j, sem_u, sem_i, sem_j):
    wid = lax.axis_index("s") * _NC + lax.axis_index("c")
    base = wid * _BPW
    lane = lax.iota(jnp.int32, 16)

    for c in range(_NCHUNK):
        pltpu.sync_copy(user_hbm.at[pl.ds(base + c * _CHUNK, _CHUNK)], idx_u.at[c])
        pltpu.sync_copy(item_i_hbm.at[pl.ds(base + c * _CHUNK, _CHUNK)], idx_i.at[c])
        pltpu.sync_copy(item_j_hbm.at[pl.ds(base + c * _CHUNK, _CHUNK)], idx_j.at[c])

    for c in range(_NCHUNK):
        cp_u = pltpu.async_copy(eu_hbm.at[idx_u.at[c]], rows_u, sem_u)
        cp_i = pltpu.async_copy(ei_hbm.at[idx_i.at[c]], rows_i, sem_i)
        cp_j = pltpu.async_copy(ei_hbm.at[idx_j.at[c]], rows_j, sem_j)
        cp_u.wait()
        cp_i.wait()
        cp_j.wait()

        def group(g, _):
            rvec = g * 16 + lane
            acc_i = [jnp.zeros((16,), jnp.float32) for _ in range(4)]
            acc_j = [jnp.zeros((16,), jnp.float32) for _ in range(4)]
            for d in range(FACTOR):
                e = (lane + d) & 63
                u = plsc.load_gather(rows_u, [rvec, e])
                vi = plsc.load_gather(rows_i, [rvec, e])
                vj = plsc.load_gather(rows_j, [rvec, e])
                k = d % 4
                acc_i[k] = acc_i[k] + u * vi
                acc_j[k] = acc_j[k] + u * vj
            out_i[pl.ds(c * _CHUNK + g * 16, 16)] = (
                (acc_i[0] + acc_i[1]) + (acc_i[2] + acc_i[3]))
            out_j[pl.ds(c * _CHUNK + g * 16, 16)] = (
                (acc_j[0] + acc_j[1]) + (acc_j[2] + acc_j[3]))
            return 0

        lax.fori_loop(0, _CHUNK // 16, group, 0)

    pltpu.sync_copy(out_i, pred_i_hbm.at[pl.ds(base, _BPW)])
    pltpu.sync_copy(out_j, pred_j_hbm.at[pl.ds(base, _BPW)])


@jax.jit
def _pairmf(user, item_i, item_j, embed_user, embed_item):
    mesh = plsc.VectorSubcoreMesh(core_axis_name="c", subcore_axis_name="s")
    f = functools.partial(
        pl.kernel, mesh=mesh,
        compiler_params=pltpu.CompilerParams(
            needs_layout_passes=False, use_tc_tiling_on_sc=True),
        out_type=(jax.ShapeDtypeStruct((BATCH,), jnp.float32),
                  jax.ShapeDtypeStruct((BATCH,), jnp.float32)),
        scratch_types=[
            pltpu.VMEM((_NCHUNK, _CHUNK), jnp.int32),
            pltpu.VMEM((_NCHUNK, _CHUNK), jnp.int32),
            pltpu.VMEM((_NCHUNK, _CHUNK), jnp.int32),
            pltpu.VMEM((_CHUNK, 2 * FACTOR), jnp.float32),
            pltpu.VMEM((_CHUNK, 2 * FACTOR), jnp.float32),
            pltpu.VMEM((_CHUNK, 2 * FACTOR), jnp.float32),
            pltpu.VMEM((_BPW,), jnp.float32),
            pltpu.VMEM((_BPW,), jnp.float32),
            pltpu.SemaphoreType.DMA,
            pltpu.SemaphoreType.DMA,
            pltpu.SemaphoreType.DMA,
        ],
    )(_pairmf_body)
    eu2 = jnp.pad(embed_user, ((0, 0), (0, FACTOR)))
    ei2 = jnp.pad(embed_item, ((0, 0), (0, FACTOR)))
    return f(user, item_i, item_j, eu2, ei2)


def kernel(user, item_i, item_j, embed_user, embed_item):
    user = user.astype(jnp.int32)
    item_i = item_i.astype(jnp.int32)
    item_j = item_j.astype(jnp.int32)
    return _pairmf(user, item_i, item_j, embed_user, embed_item)
